# baseline (speedup 1.0000x reference)
import functools

import jax
import jax.numpy as jnp
from jax import lax
from jax.experimental import pallas as pl
from jax.experimental.pallas import tpu as pltpu
from jax.experimental.pallas import tpu_sc as plsc

NC = 2
NS = 16
NW = NC * NS
LANES = 16
CHUNK = 40
ZROWS = 128


def _sc_mesh():
    return plsc.VectorSubcoreMesh(core_axis_name="c", subcore_axis_name="s")


def _make_hop_kernel(n, e, d, gather=True):
    epw = e // NW
    nchunk = epw // CHUNK
    rpt = n // NS
    nb = 8
    pf = 4
    dd = nb - pf
    assert nchunk >= 2 * nb and dd >= 1
    m_end = ((nchunk - pf) // nb) * nb

    def body(xs_hbm, row_hbm, col_hbm, out_hbm,
             ridx, cidx, rows, acc, isem, gsem, ssem):
        cid = lax.axis_index("c")
        sid = lax.axis_index("s")
        base_row = sid * rpt
        eb = (sid * NC + cid) * epw

        @pl.loop(0, CHUNK)
        def _(r):
            for j in range(d // LANES):
                rows[0, r, pl.ds(j * LANES, LANES)] = jnp.zeros(
                    (LANES,), jnp.float32)

        for m in range(rpt // CHUNK):
            pltpu.sync_copy(rows.at[0],
                            acc.at[pl.ds(base_row + m * CHUNK, CHUNK)])

        if not gather:
            @pl.loop(0, CHUNK)
            def _(r):
                for b in range(nb):
                    for j in range(d // LANES):
                        rows[b, r, pl.ds(j * LANES, LANES)] = jnp.ones(
                            (LANES,), jnp.float32)

        plsc.subcore_barrier()

        def idx_start(j, b):
            base = eb + j * CHUNK
            if gather:
                pltpu.async_copy(row_hbm.at[pl.ds(base, CHUNK)],
                                 ridx.at[b], isem.at[b])
            pltpu.async_copy(col_hbm.at[pl.ds(base, CHUNK)],
                             cidx.at[b], isem.at[b])

        def idx_wait(j, b):
            base = eb + j * CHUNK
            if gather:
                pltpu.make_async_copy(row_hbm.at[pl.ds(base, CHUNK)],
                                      ridx.at[b], isem.at[b]).wait()
            pltpu.make_async_copy(col_hbm.at[pl.ds(base, CHUNK)],
                                  cidx.at[b], isem.at[b]).wait()

        def gather_start(b):
            if gather:
                pltpu.async_copy(xs_hbm.at[ridx.at[b]], rows.at[b],
                                 gsem.at[b])

        def gather_wait(b):
            if gather:
                pltpu.make_async_copy(xs_hbm.at[ridx.at[b]], rows.at[b],
                                      gsem.at[b]).wait()

        def scat_start(b):
            pltpu.async_copy(rows.at[b], acc.at[cidx.at[b]], ssem.at[b],
                             add=True)

        def scat_wait(b):
            pltpu.make_async_copy(rows.at[b], acc.at[cidx.at[b]],
                                  ssem.at[b]).wait()

        def step(j, b, do_drain=True, more_idx=True, more_gather=True):
            gather_wait(b)
            scat_start(b)
            if do_drain:
                scat_wait((b - dd) % nb)
            if more_idx:
                idx_start(j + pf, (b + pf) % nb)
            if more_gather:
                idx_wait(j + 1, (b + 1) % nb)
                gather_start((b + 1) % nb)

        for j in range(pf):
            idx_start(j, j)
        idx_wait(0, 0)
        gather_start(0)
        for j in range(nb):
            step(j, j, do_drain=(j >= dd))

        @pl.loop(nb, m_end, step=nb)
        def _(i):
            for b in range(nb):
                step(i + b, b)

        for j in range(m_end, nchunk):
            step(j, j % nb, more_idx=(j + pf < nchunk),
                 more_gather=(j + 1 < nchunk))
        for j in range(nchunk - dd, nchunk):
            scat_wait(j % nb)

        plsc.subcore_barrier()
        pltpu.sync_copy(acc.at[pl.ds(base_row, rpt)],
                        out_hbm.at[cid, pl.ds(base_row, rpt)])

    scratch = [
        pltpu.VMEM((nb, CHUNK), jnp.int32),
        pltpu.VMEM((nb, CHUNK), jnp.int32),
        pltpu.VMEM((nb, CHUNK, d), jnp.float32),
        pltpu.VMEM_SHARED((n, d), jnp.float32),
        pltpu.SemaphoreType.DMA((nb,)),
        pltpu.SemaphoreType.DMA((nb,)),
        pltpu.SemaphoreType.DMA((nb,)),
    ]
    out_t = jax.ShapeDtypeStruct((NC, n, d), jnp.float32)

    if gather:
        @functools.partial(pl.kernel, out_type=out_t, mesh=_sc_mesh(),
                           scratch_types=scratch)
        def hop_kernel(xs_hbm, row_hbm, col_hbm, out_hbm, *s):
            body(xs_hbm, row_hbm, col_hbm, out_hbm, *s)
        return hop_kernel

    @functools.partial(pl.kernel, out_type=out_t, mesh=_sc_mesh(),
                       scratch_types=scratch)
    def deg_kernel(col_hbm, out_hbm, *s):
        body(None, None, col_hbm, out_hbm, *s)
    return deg_kernel


_BLK = 512


def _dot(a, b):
    return jax.lax.dot_general(a, b, (((1,), (0,)), ((), ())),
                               preferred_element_type=jnp.float32,
                               precision=jax.lax.Precision.HIGHEST)


def _prolog_body(p0, p1, h, w, b, acc_o, xs_o, dinv_o):
    deg = p0[:, 0:1] + p1[:, 0:1]
    dinv = jnp.where(deg > 0, jax.lax.rsqrt(deg), 0.0)
    dinvb = jnp.broadcast_to(dinv, h.shape)
    hb = h[...]
    acc_o[...] = _dot(hb, w[...]) + b[...]
    xs_o[...] = hb * dinvb
    dinv_o[...] = dinvb


def _mid_body(acc, p0, p1, dinv, w, b, acc_o, xs_o):
    x = dinv[...] * (p0[...] + p1[...])
    acc_o[...] = acc[...] + _dot(x, w[...]) + b[...]
    xs_o[...] = dinv[...] * x


def _boundary_body(acc, p0, p1, dinv, w_a, b_a, w_b, b_b, acc_o, xs_o):
    x3 = dinv[...] * (p0[...] + p1[...])
    hid = jnp.maximum(acc[...] + _dot(x3, w_a[...]) + b_a[...], 0.0)
    acc_o[...] = _dot(hid, w_b[...]) + b_b[...]
    xs_o[...] = dinv[...] * hid


def _final_body(acc, p0, p1, dinv, w, b, out_o):
    x3 = dinv[...] * (p0[...] + p1[...])
    out_o[...] = acc[...] + _dot(x3, w[...]) + b[...]


def _row_spec(n, d):
    return pl.BlockSpec((_BLK, d), lambda i: (i, 0))


def _full_spec(shape):
    nd = len(shape)
    return pl.BlockSpec(shape, lambda i: (0,) * nd)


def _tc_call(body, n, d, n_row_args, n_full_args, n_outs):
    grid = (n // _BLK,)
    in_specs = ([_row_spec(n, d)] * n_row_args
                + [_full_spec((d, d)) if k % 2 == 0 else _full_spec((1, d))
                   for k in range(n_full_args)])
    out_specs = [_row_spec(n, d)] * n_outs
    out_shape = [jax.ShapeDtypeStruct((n, d), jnp.float32)] * n_outs
    return pl.pallas_call(body, grid=grid, in_specs=in_specs,
                          out_specs=out_specs, out_shape=out_shape)


def kernel(h, edge_index, W1, b1, W2, b2):
    n, d = h.shape
    e = edge_index.shape[1]
    row = edge_index[0]
    col = edge_index[1]

    npad = -(-n // (NS * 128)) * (NS * 128)
    hp = jnp.pad(h, ((0, npad - n), (0, 0))) if npad != n else h

    deg_k = _make_hop_kernel(npad, e, d, gather=False)
    hop_k = _make_hop_kernel(npad, e, d)

    degp = deg_k(col)
    dp0, dp1 = degp[0], degp[1]

    grid = (npad // _BLK,)
    prolog = pl.pallas_call(
        _prolog_body, grid=grid,
        in_specs=[_row_spec(npad, d), _row_spec(npad, d),
                  _row_spec(npad, d), _full_spec((d, d)), _full_spec((1, d))],
        out_specs=[_row_spec(npad, d)] * 3,
        out_shape=[jax.ShapeDtypeStruct((npad, d), jnp.float32)] * 3,
    )
    acc, xs, dinv = prolog(dp0, dp1, hp, W1[0], b1[0].reshape(1, d))

    mid = _tc_call(_mid_body, npad, d, 4, 2, 2)
    boundary = _tc_call(_boundary_body, npad, d, 4, 4, 2)
    final = _tc_call(_final_body, npad, d, 4, 2, 1)

    for k in (1, 2):
        p = hop_k(xs, row, col)
        acc, xs = mid(acc, p[0], p[1], dinv, W1[k], b1[k].reshape(1, d))
    p = hop_k(xs, row, col)
    acc, xs = boundary(acc, p[0], p[1], dinv,
                       W1[3], b1[3].reshape(1, d), W2[0], b2[0].reshape(1, d))
    for k in (1, 2):
        p = hop_k(xs, row, col)
        acc, xs = mid(acc, p[0], p[1], dinv, W2[k], b2[k].reshape(1, d))
    p = hop_k(xs, row, col)
    (out,) = final(acc, p[0], p[1], dinv, W2[3], b2[3].reshape(1, d))
    return out[:n]


# speedup vs baseline: 1.3736x; 1.3736x over previous
import functools

import jax
import jax.numpy as jnp
from jax import lax
from jax.experimental import pallas as pl
from jax.experimental.pallas import tpu as pltpu
from jax.experimental.pallas import tpu_sc as plsc

NC = 2
NS = 16
NW = NC * NS
LANES = 16
CHUNK = 80
ZROWS = 128


def _sc_mesh():
    return plsc.VectorSubcoreMesh(core_axis_name="c", subcore_axis_name="s")


def _make_hop_kernel(n, e, d, gather=True):
    epw = e // NW
    nchunk = epw // CHUNK
    rpt = n // NS
    nb = 4
    pf = 2
    dd = nb - pf
    assert nchunk >= 2 * nb and dd >= 1
    m_end = ((nchunk - pf) // nb) * nb

    def body(xs_hbm, row_hbm, col_hbm, out_hbm,
             ridx, cidx, rows, acc, isem, gsem, ssem):
        cid = lax.axis_index("c")
        sid = lax.axis_index("s")
        base_row = sid * rpt
        eb = (sid * NC + cid) * epw

        @pl.loop(0, CHUNK)
        def _(r):
            for j in range(d // LANES):
                rows[0, r, pl.ds(j * LANES, LANES)] = jnp.zeros(
                    (LANES,), jnp.float32)

        for m in range(rpt // CHUNK):
            pltpu.sync_copy(rows.at[0],
                            acc.at[pl.ds(base_row + m * CHUNK, CHUNK)])

        if not gather:
            @pl.loop(0, CHUNK)
            def _(r):
                for b in range(nb):
                    for j in range(d // LANES):
                        rows[b, r, pl.ds(j * LANES, LANES)] = jnp.ones(
                            (LANES,), jnp.float32)

        plsc.subcore_barrier()

        def idx_start(j, b):
            base = eb + j * CHUNK
            if gather:
                pltpu.async_copy(row_hbm.at[pl.ds(base, CHUNK)],
                                 ridx.at[b], isem.at[b])
            pltpu.async_copy(col_hbm.at[pl.ds(base, CHUNK)],
                             cidx.at[b], isem.at[b])

        def idx_wait(j, b):
            base = eb + j * CHUNK
            if gather:
                pltpu.make_async_copy(row_hbm.at[pl.ds(base, CHUNK)],
                                      ridx.at[b], isem.at[b]).wait()
            pltpu.make_async_copy(col_hbm.at[pl.ds(base, CHUNK)],
                                  cidx.at[b], isem.at[b]).wait()

        def gather_start(b):
            if gather:
                pltpu.async_copy(xs_hbm.at[ridx.at[b]], rows.at[b],
                                 gsem.at[b])

        def gather_wait(b):
            if gather:
                pltpu.make_async_copy(xs_hbm.at[ridx.at[b]], rows.at[b],
                                      gsem.at[b]).wait()

        def scat_start(b):
            pltpu.async_copy(rows.at[b], acc.at[cidx.at[b]], ssem.at[b],
                             add=True)

        def scat_wait(b):
            pltpu.make_async_copy(rows.at[b], acc.at[cidx.at[b]],
                                  ssem.at[b]).wait()

        def step(j, b, do_drain=True, more_idx=True, more_gather=True):
            gather_wait(b)
            scat_start(b)
            if do_drain:
                scat_wait((b - dd) % nb)
            if more_idx:
                idx_start(j + pf, (b + pf) % nb)
            if more_gather:
                idx_wait(j + 1, (b + 1) % nb)
                gather_start((b + 1) % nb)

        for j in range(pf):
            idx_start(j, j)
        idx_wait(0, 0)
        gather_start(0)
        for j in range(nb):
            step(j, j, do_drain=(j >= dd))

        @pl.loop(nb, m_end, step=nb)
        def _(i):
            for b in range(nb):
                step(i + b, b)

        for j in range(m_end, nchunk):
            step(j, j % nb, more_idx=(j + pf < nchunk),
                 more_gather=(j + 1 < nchunk))
        for j in range(nchunk - dd, nchunk):
            scat_wait(j % nb)

        plsc.subcore_barrier()
        pltpu.sync_copy(acc.at[pl.ds(base_row, rpt)],
                        out_hbm.at[cid, pl.ds(base_row, rpt)])

    scratch = [
        pltpu.VMEM((nb, CHUNK), jnp.int32),
        pltpu.VMEM((nb, CHUNK), jnp.int32),
        pltpu.VMEM((nb, CHUNK, d), jnp.float32),
        pltpu.VMEM_SHARED((n, d), jnp.float32),
        pltpu.SemaphoreType.DMA((nb,)),
        pltpu.SemaphoreType.DMA((nb,)),
        pltpu.SemaphoreType.DMA((nb,)),
    ]
    out_t = jax.ShapeDtypeStruct((NC, n, d), jnp.float32)

    if gather:
        @functools.partial(pl.kernel, out_type=out_t, mesh=_sc_mesh(),
                           scratch_types=scratch)
        def hop_kernel(xs_hbm, row_hbm, col_hbm, out_hbm, *s):
            body(xs_hbm, row_hbm, col_hbm, out_hbm, *s)
        return hop_kernel

    @functools.partial(pl.kernel, out_type=out_t, mesh=_sc_mesh(),
                       scratch_types=scratch)
    def deg_kernel(col_hbm, out_hbm, *s):
        body(None, None, col_hbm, out_hbm, *s)
    return deg_kernel


_BLK = 512


def _dot(a, b):
    return jax.lax.dot_general(a, b, (((1,), (0,)), ((), ())),
                               preferred_element_type=jnp.float32,
                               precision=jax.lax.Precision.HIGHEST)


def _prolog_body(p0, p1, h, w, b, acc_o, xs_o, dinv_o):
    deg = p0[:, 0:1] + p1[:, 0:1]
    dinv = jnp.where(deg > 0, jax.lax.rsqrt(deg), 0.0)
    dinvb = jnp.broadcast_to(dinv, h.shape)
    hb = h[...]
    acc_o[...] = _dot(hb, w[...]) + b[...]
    xs_o[...] = hb * dinvb
    dinv_o[...] = dinvb


def _mid_body(acc, p0, p1, dinv, w, b, acc_o, xs_o):
    x = dinv[...] * (p0[...] + p1[...])
    acc_o[...] = acc[...] + _dot(x, w[...]) + b[...]
    xs_o[...] = dinv[...] * x


def _boundary_body(acc, p0, p1, dinv, w_a, b_a, w_b, b_b, acc_o, xs_o):
    x3 = dinv[...] * (p0[...] + p1[...])
    hid = jnp.maximum(acc[...] + _dot(x3, w_a[...]) + b_a[...], 0.0)
    acc_o[...] = _dot(hid, w_b[...]) + b_b[...]
    xs_o[...] = dinv[...] * hid


def _final_body(acc, p0, p1, dinv, w, b, out_o):
    x3 = dinv[...] * (p0[...] + p1[...])
    out_o[...] = acc[...] + _dot(x3, w[...]) + b[...]


def _row_spec(n, d):
    return pl.BlockSpec((_BLK, d), lambda i: (i, 0))


def _full_spec(shape):
    nd = len(shape)
    return pl.BlockSpec(shape, lambda i: (0,) * nd)


def _tc_call(body, n, d, n_row_args, n_full_args, n_outs):
    grid = (n // _BLK,)
    in_specs = ([_row_spec(n, d)] * n_row_args
                + [_full_spec((d, d)) if k % 2 == 0 else _full_spec((1, d))
                   for k in range(n_full_args)])
    out_specs = [_row_spec(n, d)] * n_outs
    out_shape = [jax.ShapeDtypeStruct((n, d), jnp.float32)] * n_outs
    return pl.pallas_call(body, grid=grid, in_specs=in_specs,
                          out_specs=out_specs, out_shape=out_shape)


def kernel(h, edge_index, W1, b1, W2, b2):
    n, d = h.shape
    e = edge_index.shape[1]
    row = edge_index[0]
    col = edge_index[1]

    npad = -(-n // (NS * 128)) * (NS * 128)
    hp = jnp.pad(h, ((0, npad - n), (0, 0))) if npad != n else h

    deg_k = _make_hop_kernel(npad, e, d, gather=False)
    hop_k = _make_hop_kernel(npad, e, d)

    degp = deg_k(col)
    dp0, dp1 = degp[0], degp[1]

    grid = (npad // _BLK,)
    prolog = pl.pallas_call(
        _prolog_body, grid=grid,
        in_specs=[_row_spec(npad, d), _row_spec(npad, d),
                  _row_spec(npad, d), _full_spec((d, d)), _full_spec((1, d))],
        out_specs=[_row_spec(npad, d)] * 3,
        out_shape=[jax.ShapeDtypeStruct((npad, d), jnp.float32)] * 3,
    )
    acc, xs, dinv = prolog(dp0, dp1, hp, W1[0], b1[0].reshape(1, d))

    mid = _tc_call(_mid_body, npad, d, 4, 2, 2)
    boundary = _tc_call(_boundary_body, npad, d, 4, 4, 2)
    final = _tc_call(_final_body, npad, d, 4, 2, 1)

    for k in (1, 2):
        p = hop_k(xs, row, col)
        acc, xs = mid(acc, p[0], p[1], dinv, W1[k], b1[k].reshape(1, d))
    p = hop_k(xs, row, col)
    acc, xs = boundary(acc, p[0], p[1], dinv,
                       W1[3], b1[3].reshape(1, d), W2[0], b2[0].reshape(1, d))
    for k in (1, 2):
        p = hop_k(xs, row, col)
        acc, xs = mid(acc, p[0], p[1], dinv, W2[k], b2[k].reshape(1, d))
    p = hop_k(xs, row, col)
    (out,) = final(acc, p[0], p[1], dinv, W2[3], b2[3].reshape(1, d))
    return out[:n]
